# baseline (speedup 1.0000x reference)
import functools

import jax
import jax.numpy as jnp
from jax import lax
from jax.experimental import pallas as pl
from jax.experimental.pallas import tpu as pltpu
from jax.experimental.pallas import tpu_sc as plsc

N = 10000
E = 160000
D_IN = 128
D_EDGE = 16
H = 64
K = 128
PH = 512
STEPS = 6

NC = 2
NS = 16
NW = NC * NS
CH = 128
EPW = 5120
E_PAD = NW * EPW
NCHUNK = EPW // CH
N_PAD = 10112
RPT = N_PAD // NS

EB = 512
EGB = 2000
NB = 1000

_BF = jnp.bfloat16
_F32 = jnp.float32


def _bdot(a_bf16, b_bf16):
    return jnp.dot(a_bf16, b_bf16, preferred_element_type=_F32)


def _rnd(x_f32):
    return x_f32.astype(_BF).astype(_F32)


def _full(shape):
    return pl.BlockSpec(shape, lambda i: (0,) * len(shape))


def _proj_body(x_ref, w0, w1, w2, w3, b0, b1, b2, b3, o_ref, obf_ref):
    h = x_ref[...]
    for w, b in ((w0, b0), (w1, b1), (w2, b2), (w3, b3)):
        h = jnp.maximum(_bdot(h.astype(_BF), w[...]) + b[...], 0.0)
    z = jnp.zeros_like(h)
    o_ref[...] = jnp.concatenate([h, z], axis=1)
    obf_ref[...] = jnp.concatenate([_rnd(h), z], axis=1)


def _proj(x, ws, bs):
    return pl.pallas_call(
        _proj_body,
        grid=(N // NB,),
        in_specs=[pl.BlockSpec((NB, D_IN), lambda i: (i, 0)),
                  _full((D_IN, H)), _full((H, H)), _full((H, H)), _full((H, H)),
                  _full((1, H)), _full((1, H)), _full((1, H)), _full((1, H))],
        out_specs=[pl.BlockSpec((NB, K), lambda i: (i, 0)),
                   pl.BlockSpec((NB, K), lambda i: (i, 0))],
        out_shape=[jax.ShapeDtypeStruct((N, K), _F32),
                   jax.ShapeDtypeStruct((N, K), _F32)],
    )(x, *ws, *bs)


def _edgeg_body(a_ref, w0, w1, w2, b0, b1, b2, o_ref):
    h = a_ref[...]
    for w, b in ((w0, b0), (w1, b1), (w2, b2)):
        h = jnp.maximum(_bdot(h.astype(_BF), w[...]) + b[...], 0.0)
    o_ref[...] = h.astype(_BF)


def _edgeg(ea_pad, ws, bs):
    return pl.pallas_call(
        _edgeg_body,
        grid=(E // EGB,),
        in_specs=[pl.BlockSpec((EGB, K), lambda i: (i, 0)),
                  _full((K, K)), _full((K, K)), _full((K, K)),
                  _full((1, K)), _full((1, K)), _full((1, K))],
        out_specs=pl.BlockSpec((EGB, K), lambda i: (i, 0)),
        out_shape=jax.ShapeDtypeStruct((E, K), _BF),
    )(ea_pad, *ws, *bs)


def _msg_body(hs_ref, g_ref, v_ref, b3_ref, o_ref):
    hs = hs_ref[...][:, :H]
    w = _bdot(g_ref[...], v_ref[...]) + b3_ref[...]
    w = _rnd(w)
    acc = jnp.zeros((EB, H), _F32)
    for i in range(H):
        acc = acc + hs[:, i:i + 1] * w[:, i * H:(i + 1) * H]
    o_ref[...] = jnp.concatenate([acc, jnp.zeros_like(acc)], axis=1)


def _msg(hsrc, g_pad, v_bf, b3row):
    return pl.pallas_call(
        _msg_body,
        grid=(E_PAD // EB,),
        in_specs=[pl.BlockSpec((EB, K), lambda i: (i, 0)),
                  pl.BlockSpec((EB, K), lambda i: (i, 0)),
                  _full((K, H * H)), _full((1, H * H))],
        out_specs=pl.BlockSpec((EB, K), lambda i: (i, 0)),
        out_shape=jax.ShapeDtypeStruct((E_PAD, K), _F32),
    )(hsrc, g_pad, v_bf, b3row)


def _sigmoid(x):
    return 1.0 / (1.0 + jnp.exp(-x))


def _gru_body(p_ref, hid_ref, wih, whh, bih, bhh, cb, o_ref, obf_ref):
    p = p_ref[...]
    hid = hid_ref[...][:, :H]
    a = jnp.maximum(p[0, :, :H] + p[1, :, :H] + cb[...], 0.0)
    gi = _bdot(a.astype(_BF), wih[...]) + bih[...]
    gh = _bdot(hid.astype(_BF), whh[...]) + bhh[...]
    r = _sigmoid(gi[:, :H] + gh[:, :H])
    z = _sigmoid(gi[:, H:2 * H] + gh[:, H:2 * H])
    n = jnp.tanh(gi[:, 2 * H:] + r * gh[:, 2 * H:])
    new = (1.0 - z) * n + z * hid
    zz = jnp.zeros_like(new)
    o_ref[...] = jnp.concatenate([new, zz], axis=1)
    obf_ref[...] = jnp.concatenate([_rnd(new), zz], axis=1)


def _gru(parts, hid, wih, whh, bih, bhh, cb):
    return pl.pallas_call(
        _gru_body,
        grid=(N // NB,),
        in_specs=[pl.BlockSpec((2, NB, K), lambda i: (0, i, 0)),
                  pl.BlockSpec((NB, K), lambda i: (i, 0)),
                  _full((H, 3 * H)), _full((H, 3 * H)),
                  _full((1, 3 * H)), _full((1, 3 * H)), _full((1, H))],
        out_specs=[pl.BlockSpec((NB, K), lambda i: (i, 0)),
                   pl.BlockSpec((NB, K), lambda i: (i, 0))],
        out_shape=[jax.ShapeDtypeStruct((N, K), _F32),
                   jax.ShapeDtypeStruct((N, K), _F32)],
    )(parts, hid, wih, whh, bih, bhh, cb)


def _pred_body(h_ref, w0, w1, w2, b0, b1, b2, o_ref):
    h = h_ref[...][:, :H]
    h = jnp.maximum(_bdot(h.astype(_BF), w0[...]) + b0[...], 0.0)
    h = jnp.maximum(_bdot(h.astype(_BF), w1[...]) + b1[...], 0.0)
    o_ref[...] = _bdot(h.astype(_BF), w2[...]) + b2[...]


def _pred(h, w0, w1, w2p, b0, b1, b2p):
    return pl.pallas_call(
        _pred_body,
        grid=(N // NB,),
        in_specs=[pl.BlockSpec((NB, K), lambda i: (i, 0)),
                  _full((H, PH)), _full((PH, PH)), _full((PH, K)),
                  _full((1, PH)), _full((1, PH)), _full((1, K))],
        out_specs=pl.BlockSpec((NB, K), lambda i: (i, 0)),
        out_shape=jax.ShapeDtypeStruct((N, K), _F32),
    )(h, w0, w1, w2p, b0, b1, b2p)


@functools.cache
def _sc_kernels():
    mesh = plsc.VectorSubcoreMesh(core_axis_name="c", subcore_axis_name="s",
                                  num_cores=NC, num_subcores=NS)

    @functools.partial(
        pl.kernel,
        out_type=jax.ShapeDtypeStruct((E_PAD, K), _F32),
        mesh=mesh,
        scratch_types=[pltpu.VMEM((NCHUNK, CH), jnp.int32),
                       pltpu.VMEM((CH, K), _F32),
                       pltpu.SemaphoreType.DMA],
    )
    def sc_gather(nf_hbm, idx_hbm, out_hbm, idx_v, rows_v, sem):
        wid = lax.axis_index("c") * NS + lax.axis_index("s")
        pltpu.sync_copy(idx_hbm.at[wid], idx_v)

        def body(j, carry):
            pltpu.async_copy(nf_hbm.at[idx_v.at[j]], rows_v, sem).wait()
            pltpu.sync_copy(rows_v, out_hbm.at[pl.ds(wid * EPW + j * CH, CH)])
            return carry

        lax.fori_loop(0, NCHUNK, body, 0)

    @functools.partial(
        pl.kernel,
        out_type=jax.ShapeDtypeStruct((NC, N_PAD, K), _F32),
        mesh=mesh,
        scratch_types=[pltpu.VMEM((NCHUNK, CH), jnp.int32),
                       pltpu.VMEM((CH, K), _F32),
                       pltpu.VMEM_SHARED((N_PAD, K), _F32)],
    )
    def sc_scatter(msg_hbm, dst_hbm, zeros_hbm, out_hbm, idx_v, rows_v, acc_sh):
        c = lax.axis_index("c")
        s = lax.axis_index("s")
        wid = c * NS + s
        pltpu.sync_copy(zeros_hbm.at[pl.ds(s * RPT, RPT)],
                        acc_sh.at[pl.ds(s * RPT, RPT)])
        plsc.subcore_barrier()
        pltpu.sync_copy(dst_hbm.at[wid], idx_v)

        def body(j, carry):
            pltpu.sync_copy(msg_hbm.at[pl.ds(wid * EPW + j * CH, CH)], rows_v)
            pltpu.sync_copy(rows_v, acc_sh.at[idx_v.at[j]], add=True)
            return carry

        lax.fori_loop(0, NCHUNK, body, 0)
        plsc.subcore_barrier()
        pltpu.sync_copy(acc_sh.at[pl.ds(s * RPT, RPT)],
                        out_hbm.at[c, pl.ds(s * RPT, RPT)])

    return sc_gather, sc_scatter


def kernel(x, edge_index, edge_attr, params):
    p = params
    src = edge_index[0]
    dst = edge_index[1]
    pad = E_PAD - E
    src3 = jnp.concatenate(
        [src, jnp.zeros((pad,), jnp.int32)]).reshape(NW, NCHUNK, CH)
    dst3 = jnp.concatenate(
        [dst, jnp.full((pad,), N, jnp.int32)]).reshape(NW, NCHUNK, CH)

    h, h_bf = _proj(x, [w.astype(_BF) for w in p['proj_W']],
                    [b.reshape(1, -1) for b in p['proj_b']])

    ea_pad = jnp.pad(edge_attr, ((0, 0), (0, K - D_EDGE)))
    w0_pad = jnp.pad(p['edge_W'][0], ((0, K - D_EDGE), (0, 0)))
    g = _edgeg(ea_pad,
               [w0_pad.astype(_BF), p['edge_W'][1].astype(_BF),
                p['edge_W'][2].astype(_BF)],
               [b.reshape(1, -1) for b in p['edge_b'][:3]])
    g_pad = jnp.concatenate([g, jnp.zeros((pad, K), _BF)])

    v_bf = p['edge_W'][3].astype(_BF)
    b3row = p['edge_b'][3].reshape(1, H * H)
    zeros_buf = jnp.zeros((N_PAD, K), _F32)

    gru_w = (p['gru_Wih'].astype(_BF), p['gru_Whh'].astype(_BF),
             p['gru_bih'].reshape(1, -1), p['gru_bhh'].reshape(1, -1),
             p['conv_b'].reshape(1, -1))

    sc_gather, sc_scatter = _sc_kernels()
    hid = h
    nf_bf = h_bf
    for _ in range(STEPS):
        hsrc = sc_gather(nf_bf, src3)
        msg = _msg(hsrc, g_pad, v_bf, b3row)
        parts = sc_scatter(msg, dst3, zeros_buf)
        hid, nf_bf = _gru(parts, hid, *gru_w)

    w2p = jnp.pad(p['pred_W'][2], ((0, 0), (0, K - 1)))
    b2p = jnp.pad(p['pred_b'][2], ((0, K - 1),)).reshape(1, -1)
    out = _pred(hid, p['pred_W'][0].astype(_BF), p['pred_W'][1].astype(_BF),
                w2p.astype(_BF), p['pred_b'][0].reshape(1, -1),
                p['pred_b'][1].reshape(1, -1), b2p)
    return out[:, 0]


# speedup vs baseline: 2.3455x; 2.3455x over previous
import functools

import jax
import jax.numpy as jnp
from jax import lax
from jax.experimental import pallas as pl
from jax.experimental.pallas import tpu as pltpu
from jax.experimental.pallas import tpu_sc as plsc

N = 10000
E = 160000
D_IN = 128
D_EDGE = 16
H = 64
K = 128
PH = 512
STEPS = 6

NC = 2
NS = 16
NW = NC * NS
CH = 128
EPW = 5120
E_PAD = NW * EPW
NCHUNK = EPW // CH
N_PAD = 10112
RPT = N_PAD // NS

EB = 512
EGB = 2000
NB = 1000

_BF = jnp.bfloat16
_F32 = jnp.float32


def _bdot(a_bf16, b_bf16):
    return jnp.dot(a_bf16, b_bf16, preferred_element_type=_F32)


def _rnd(x_f32):
    return x_f32.astype(_BF).astype(_F32)


def _full(shape):
    return pl.BlockSpec(shape, lambda i: (0,) * len(shape))


def _proj_body(x_ref, w0, w1, w2, w3, b0, b1, b2, b3, o_ref, obf_ref):
    h = x_ref[...]
    for w, b in ((w0, b0), (w1, b1), (w2, b2), (w3, b3)):
        h = jnp.maximum(_bdot(h.astype(_BF), w[...]) + b[...], 0.0)
    z = jnp.zeros_like(h)
    o_ref[...] = jnp.concatenate([h, z], axis=1)
    obf_ref[...] = jnp.concatenate([_rnd(h), z], axis=1)


def _proj(x, ws, bs):
    return pl.pallas_call(
        _proj_body,
        grid=(N // NB,),
        in_specs=[pl.BlockSpec((NB, D_IN), lambda i: (i, 0)),
                  _full((D_IN, H)), _full((H, H)), _full((H, H)), _full((H, H)),
                  _full((1, H)), _full((1, H)), _full((1, H)), _full((1, H))],
        out_specs=[pl.BlockSpec((NB, K), lambda i: (i, 0)),
                   pl.BlockSpec((NB, K), lambda i: (i, 0))],
        out_shape=[jax.ShapeDtypeStruct((N, K), _F32),
                   jax.ShapeDtypeStruct((N, K), _F32)],
    )(x, *ws, *bs)


def _edgeg_body(a_ref, w0, w1, w2, b0, b1, b2, o_ref):
    h = a_ref[...]
    for w, b in ((w0, b0), (w1, b1), (w2, b2)):
        h = jnp.maximum(_bdot(h.astype(_BF), w[...]) + b[...], 0.0)
    o_ref[...] = h.astype(_BF)


def _edgeg(ea_pad, ws, bs):
    return pl.pallas_call(
        _edgeg_body,
        grid=(E // EGB,),
        in_specs=[pl.BlockSpec((EGB, K), lambda i: (i, 0)),
                  _full((K, K)), _full((K, K)), _full((K, K)),
                  _full((1, K)), _full((1, K)), _full((1, K))],
        out_specs=pl.BlockSpec((EGB, K), lambda i: (i, 0)),
        out_shape=jax.ShapeDtypeStruct((E, K), _BF),
    )(ea_pad, *ws, *bs)


def _msg_body(hs_ref, g_ref, v_ref, b3_ref, bmat_ref, o_ref):
    hs = hs_ref[...][:, :H]
    w = _bdot(g_ref[...], v_ref[...]) + b3_ref[...]
    w = _rnd(w)
    hs_bcast = _bdot(hs.astype(_BF), bmat_ref[...])
    acc = jnp.zeros((EB, K), _F32)
    for j in range(H * H // K):
        acc = acc + hs_bcast[:, j * K:(j + 1) * K] * w[:, j * K:(j + 1) * K]
    msg = acc[:, :H] + acc[:, H:]
    o_ref[...] = jnp.concatenate([msg, jnp.zeros_like(msg)], axis=1)


def _msg(hsrc, g_pad, v_bf, b3row, bmat):
    return pl.pallas_call(
        _msg_body,
        grid=(E_PAD // EB,),
        in_specs=[pl.BlockSpec((EB, K), lambda i: (i, 0)),
                  pl.BlockSpec((EB, K), lambda i: (i, 0)),
                  _full((K, H * H)), _full((1, H * H)), _full((H, H * H))],
        out_specs=pl.BlockSpec((EB, K), lambda i: (i, 0)),
        out_shape=jax.ShapeDtypeStruct((E_PAD, K), _F32),
    )(hsrc, g_pad, v_bf, b3row, bmat)


def _sigmoid(x):
    return 1.0 / (1.0 + jnp.exp(-x))


def _gru_body(p_ref, hid_ref, wih, whh, bih, bhh, cb, o_ref, obf_ref):
    p = p_ref[...]
    hid = hid_ref[...][:, :H]
    a = jnp.maximum(p[0, :, :H] + p[1, :, :H] + cb[...], 0.0)
    gi = _bdot(a.astype(_BF), wih[...]) + bih[...]
    gh = _bdot(hid.astype(_BF), whh[...]) + bhh[...]
    r = _sigmoid(gi[:, :H] + gh[:, :H])
    z = _sigmoid(gi[:, H:2 * H] + gh[:, H:2 * H])
    n = jnp.tanh(gi[:, 2 * H:] + r * gh[:, 2 * H:])
    new = (1.0 - z) * n + z * hid
    zz = jnp.zeros_like(new)
    o_ref[...] = jnp.concatenate([new, zz], axis=1)
    obf_ref[...] = jnp.concatenate([_rnd(new), zz], axis=1)


def _gru(parts, hid, wih, whh, bih, bhh, cb):
    return pl.pallas_call(
        _gru_body,
        grid=(N // NB,),
        in_specs=[pl.BlockSpec((2, NB, K), lambda i: (0, i, 0)),
                  pl.BlockSpec((NB, K), lambda i: (i, 0)),
                  _full((H, 3 * H)), _full((H, 3 * H)),
                  _full((1, 3 * H)), _full((1, 3 * H)), _full((1, H))],
        out_specs=[pl.BlockSpec((NB, K), lambda i: (i, 0)),
                   pl.BlockSpec((NB, K), lambda i: (i, 0))],
        out_shape=[jax.ShapeDtypeStruct((N, K), _F32),
                   jax.ShapeDtypeStruct((N, K), _F32)],
    )(parts, hid, wih, whh, bih, bhh, cb)


def _pred_body(h_ref, w0, w1, w2, b0, b1, b2, o_ref):
    h = h_ref[...][:, :H]
    h = jnp.maximum(_bdot(h.astype(_BF), w0[...]) + b0[...], 0.0)
    h = jnp.maximum(_bdot(h.astype(_BF), w1[...]) + b1[...], 0.0)
    o_ref[...] = _bdot(h.astype(_BF), w2[...]) + b2[...]


def _pred(h, w0, w1, w2p, b0, b1, b2p):
    return pl.pallas_call(
        _pred_body,
        grid=(N // NB,),
        in_specs=[pl.BlockSpec((NB, K), lambda i: (i, 0)),
                  _full((H, PH)), _full((PH, PH)), _full((PH, K)),
                  _full((1, PH)), _full((1, PH)), _full((1, K))],
        out_specs=pl.BlockSpec((NB, K), lambda i: (i, 0)),
        out_shape=jax.ShapeDtypeStruct((N, K), _F32),
    )(h, w0, w1, w2p, b0, b1, b2p)


@functools.cache
def _sc_kernels():
    mesh = plsc.VectorSubcoreMesh(core_axis_name="c", subcore_axis_name="s",
                                  num_cores=NC, num_subcores=NS)

    @functools.partial(
        pl.kernel,
        out_type=jax.ShapeDtypeStruct((E_PAD, K), _F32),
        mesh=mesh,
        scratch_types=[pltpu.VMEM((NCHUNK, CH), jnp.int32),
                       pltpu.VMEM((CH, K), _F32),
                       pltpu.SemaphoreType.DMA],
    )
    def sc_gather(nf_hbm, idx_hbm, out_hbm, idx_v, rows_v, sem):
        wid = lax.axis_index("c") * NS + lax.axis_index("s")
        pltpu.sync_copy(idx_hbm.at[wid], idx_v)

        def body(j, carry):
            pltpu.async_copy(nf_hbm.at[idx_v.at[j]], rows_v, sem).wait()
            pltpu.sync_copy(rows_v, out_hbm.at[pl.ds(wid * EPW + j * CH, CH)])
            return carry

        lax.fori_loop(0, NCHUNK, body, 0)

    @functools.partial(
        pl.kernel,
        out_type=jax.ShapeDtypeStruct((NC, N_PAD, K), _F32),
        mesh=mesh,
        scratch_types=[pltpu.VMEM((NCHUNK, CH), jnp.int32),
                       pltpu.VMEM((CH, K), _F32),
                       pltpu.VMEM_SHARED((N_PAD, K), _F32)],
    )
    def sc_scatter(msg_hbm, dst_hbm, zeros_hbm, out_hbm, idx_v, rows_v, acc_sh):
        c = lax.axis_index("c")
        s = lax.axis_index("s")
        wid = c * NS + s
        pltpu.sync_copy(zeros_hbm.at[pl.ds(s * RPT, RPT)],
                        acc_sh.at[pl.ds(s * RPT, RPT)])
        plsc.subcore_barrier()
        pltpu.sync_copy(dst_hbm.at[wid], idx_v)

        def body(j, carry):
            pltpu.sync_copy(msg_hbm.at[pl.ds(wid * EPW + j * CH, CH)], rows_v)
            pltpu.sync_copy(rows_v, acc_sh.at[idx_v.at[j]], add=True)
            return carry

        lax.fori_loop(0, NCHUNK, body, 0)
        plsc.subcore_barrier()
        pltpu.sync_copy(acc_sh.at[pl.ds(s * RPT, RPT)],
                        out_hbm.at[c, pl.ds(s * RPT, RPT)])

    return sc_gather, sc_scatter


def kernel(x, edge_index, edge_attr, params):
    p = params
    src = edge_index[0]
    dst = edge_index[1]
    pad = E_PAD - E
    src3 = jnp.concatenate(
        [src, jnp.zeros((pad,), jnp.int32)]).reshape(NW, NCHUNK, CH)
    dst3 = jnp.concatenate(
        [dst, jnp.full((pad,), N, jnp.int32)]).reshape(NW, NCHUNK, CH)

    h, h_bf = _proj(x, [w.astype(_BF) for w in p['proj_W']],
                    [b.reshape(1, -1) for b in p['proj_b']])

    ea_pad = jnp.pad(edge_attr, ((0, 0), (0, K - D_EDGE)))
    w0_pad = jnp.pad(p['edge_W'][0], ((0, K - D_EDGE), (0, 0)))
    g = _edgeg(ea_pad,
               [w0_pad.astype(_BF), p['edge_W'][1].astype(_BF),
                p['edge_W'][2].astype(_BF)],
               [b.reshape(1, -1) for b in p['edge_b'][:3]])
    g_pad = jnp.concatenate([g, jnp.zeros((pad, K), _BF)])

    v_bf = p['edge_W'][3].astype(_BF)
    b3row = p['edge_b'][3].reshape(1, H * H)
    bmat = jnp.repeat(jnp.eye(H, dtype=_BF), H, axis=1)
    zeros_buf = jnp.zeros((N_PAD, K), _F32)

    gru_w = (p['gru_Wih'].astype(_BF), p['gru_Whh'].astype(_BF),
             p['gru_bih'].reshape(1, -1), p['gru_bhh'].reshape(1, -1),
             p['conv_b'].reshape(1, -1))

    sc_gather, sc_scatter = _sc_kernels()
    hid = h
    nf_bf = h_bf
    for _ in range(STEPS):
        hsrc = sc_gather(nf_bf, src3)
        msg = _msg(hsrc, g_pad, v_bf, b3row, bmat)
        parts = sc_scatter(msg, dst3, zeros_buf)
        hid, nf_bf = _gru(parts, hid, *gru_w)

    w2p = jnp.pad(p['pred_W'][2], ((0, 0), (0, K - 1)))
    b2p = jnp.pad(p['pred_b'][2], ((0, K - 1),)).reshape(1, -1)
    out = _pred(hid, p['pred_W'][0].astype(_BF), p['pred_W'][1].astype(_BF),
                w2p.astype(_BF), p['pred_b'][0].reshape(1, -1),
                p['pred_b'][1].reshape(1, -1), b2p)
    return out[:, 0]
